# baseline (speedup 1.0000x reference)
import functools

import jax
import jax.numpy as jnp
from jax import lax
from jax.experimental import pallas as pl
from jax.experimental.pallas import tpu as pltpu
from jax.experimental.pallas import tpu_sc as plsc

NC, NS, L = 2, 16, 16
NW = NC * NS
B = 16384
D = 64
VOCAB = 1000000
K = 5
BPW = B // NW
CH = 128
NCHUNK = BPW // CH
GRP = CH // L


def _logsig(x):
    e = jnp.exp(-jnp.abs(x))
    t = e / (2.0 + e)
    t2 = t * t
    p = 1.0 / 9.0
    for c in (1.0 / 7.0, 1.0 / 5.0, 1.0 / 3.0, 1.0):
        p = p * t2 + c
    return jnp.minimum(x, 0.0) - 2.0 * t * p


_mesh = plsc.VectorSubcoreMesh(core_axis_name="c", subcore_axis_name="s")


@functools.partial(
    pl.kernel,
    mesh=_mesh,
    compiler_params=pltpu.CompilerParams(
        needs_layout_passes=False, use_tc_tiling_on_sc=False),
    out_type=jax.ShapeDtypeStruct((NW // 8, 8 * L), jnp.float32),
    scratch_types=[
        pltpu.VMEM((BPW,), jnp.int32),
        pltpu.VMEM((BPW,), jnp.int32),
        pltpu.VMEM((BPW * K,), jnp.int32),
        pltpu.VMEM((CH, D), jnp.float32),
        pltpu.VMEM((CH, D), jnp.float32),
        pltpu.VMEM((CH, D), jnp.float32),
        pltpu.VMEM((CH, D), jnp.float32),
        pltpu.VMEM((CH * K, D), jnp.float32),
        pltpu.VMEM((CH * K, D), jnp.float32),
        pltpu.VMEM((L,), jnp.float32),
        pltpu.SemaphoreType.DMA,
        pltpu.SemaphoreType.DMA,
    ],
)
def _sc_loss(center_hbm, pos_hbm, negf_hbm, inemb_hbm, outemb_hbm,
             out_hbm, ci, pi, ni, rc0, rc1, rp0, rp1, rn0, rn1,
             accv, sem0, sem1):
    rc = (rc0, rc1)
    rp = (rp0, rp1)
    rn = (rn0, rn1)
    wid = lax.axis_index("s") * NC + lax.axis_index("c")
    base = wid * BPW

    pltpu.sync_copy(center_hbm.at[pl.ds(base, BPW)], ci)
    pltpu.sync_copy(pos_hbm.at[pl.ds(base, BPW)], pi)
    pltpu.sync_copy(negf_hbm.at[pl.ds(base * K, BPW * K)], ni)

    sems = (sem0, sem1)

    def issue(c):
        s = c % 2
        hs = [
            pltpu.async_copy(inemb_hbm.at[ci.at[pl.ds(c * CH, CH)]],
                             rc[s], sems[s]),
            pltpu.async_copy(outemb_hbm.at[pi.at[pl.ds(c * CH, CH)]],
                             rp[s], sems[s]),
        ]
        for j in range(K):
            hs.append(pltpu.async_copy(
                outemb_hbm.at[ni.at[pl.ds(c * CH * K + j * CH, CH)]],
                rn[s].at[pl.ds(j * CH, CH)], sems[s]))
        return hs

    handles = {0: issue(0)}
    acc = jnp.zeros((L,), jnp.float32)

    for c in range(NCHUNK):
        if c + 1 < NCHUNK:
            handles[c + 1] = issue(c + 1)
        for h in handles.pop(c):
            h.wait()
        s = c % 2
        rc_s, rp_s, rn_s = rc[s], rp[s], rn[s]

        def gbody(g, acc):
            rows = g * L + lax.iota(jnp.int32, L)
            rows5 = rows * K

            def dbody(d, carry):
                pos, n0, n1, n2, n3, n4 = carry
                dv = jnp.full((L,), d, jnp.int32)
                cd = plsc.load_gather(rc_s, [rows, dv])
                pd = plsc.load_gather(rp_s, [rows, dv])
                pos = pos + cd * pd
                ns = []
                for k, nk in enumerate((n0, n1, n2, n3, n4)):
                    nd = plsc.load_gather(rn_s, [rows5 + k, dv])
                    ns.append(nk + cd * nd)
                return (pos, ns[0], ns[1], ns[2], ns[3], ns[4])

            z = jnp.zeros((L,), jnp.float32)
            pos, n0, n1, n2, n3, n4 = lax.fori_loop(
                0, D, dbody, (z, z, z, z, z, z))
            tot = _logsig(pos)
            for nk in (n0, n1, n2, n3, n4):
                tot = tot + _logsig(-nk)
            return acc - tot

        acc = lax.fori_loop(0, GRP, gbody, acc)

    accv[...] = acc
    pltpu.sync_copy(accv, out_hbm.at[wid // 8, pl.ds((wid % 8) * L, L)])


TRB = 2048


def _transpose_body(a_ref, b_ref, oa_ref, ob_ref):
    oa_ref[...] = a_ref[...].T
    ob_ref[...] = b_ref[...].T


_transpose = pl.pallas_call(
    _transpose_body,
    grid=(pl.cdiv(VOCAB, TRB),),
    in_specs=[
        pl.BlockSpec((D, TRB), lambda i: (0, i)),
        pl.BlockSpec((D, TRB), lambda i: (0, i)),
    ],
    out_specs=[
        pl.BlockSpec((TRB, D), lambda i: (i, 0)),
        pl.BlockSpec((TRB, D), lambda i: (i, 0)),
    ],
    out_shape=[
        jax.ShapeDtypeStruct((VOCAB, D), jnp.float32),
        jax.ShapeDtypeStruct((VOCAB, D), jnp.float32),
    ],
)


def _sum_body(x_ref, o_ref):
    o_ref[...] = jnp.full((1, 1), jnp.sum(x_ref[...]) * (1.0 / B),
                          jnp.float32)


_sum = pl.pallas_call(
    _sum_body,
    out_shape=jax.ShapeDtypeStruct((1, 1), jnp.float32),
)


def kernel(center, pos_context, neg_context, in_embed, out_embed):
    center = center.astype(jnp.int32)
    pos_context = pos_context.astype(jnp.int32)
    neg_flat = neg_context.astype(jnp.int32).reshape(-1)
    r_in, r_out = _transpose(in_embed.T, out_embed.T)
    partials = _sc_loss(center, pos_context, neg_flat, r_in, r_out)
    return _sum(partials)[0, 0]


# speedup vs baseline: 2.7315x; 2.7315x over previous
import functools

import jax
import jax.numpy as jnp
from jax import lax
from jax.experimental import pallas as pl
from jax.experimental.pallas import tpu as pltpu
from jax.experimental.pallas import tpu_sc as plsc

NC, NS, L = 2, 16, 16
NW = NC * NS
B = 16384
D = 64
VOCAB = 1000000
K = 5
BPW = B // NW
CH = 64
NCHUNK = BPW // CH
GRP = CH // L


def _logsig(x):
    e = jnp.exp(-jnp.abs(x))
    t = e / (2.0 + e)
    t2 = t * t
    p = 1.0 / 9.0
    for c in (1.0 / 7.0, 1.0 / 5.0, 1.0 / 3.0, 1.0):
        p = p * t2 + c
    return jnp.minimum(x, 0.0) - 2.0 * t * p


_mesh = plsc.VectorSubcoreMesh(core_axis_name="c", subcore_axis_name="s")


@functools.partial(
    pl.kernel,
    mesh=_mesh,
    compiler_params=pltpu.CompilerParams(
        needs_layout_passes=False, use_tc_tiling_on_sc=False),
    out_type=jax.ShapeDtypeStruct((NW // 8, 8 * L), jnp.float32),
    scratch_types=[
        pltpu.VMEM((BPW,), jnp.int32),
        pltpu.VMEM((BPW,), jnp.int32),
        pltpu.VMEM((BPW * K,), jnp.int32),
        pltpu.VMEM((BPW,), jnp.int32),
        pltpu.VMEM((BPW,), jnp.int32),
        pltpu.VMEM((BPW * K,), jnp.int32),
        pltpu.VMEM((CH, 2 * D), jnp.float32),
        pltpu.VMEM((CH, 2 * D), jnp.float32),
        pltpu.VMEM((CH, 2 * D), jnp.float32),
        pltpu.VMEM((CH, 2 * D), jnp.float32),
        pltpu.VMEM((CH * K, 2 * D), jnp.float32),
        pltpu.VMEM((CH * K, 2 * D), jnp.float32),
        pltpu.VMEM((L,), jnp.float32),
        pltpu.SemaphoreType.DMA,
        pltpu.SemaphoreType.DMA,
    ],
)
def _sc_loss(center_hbm, pos_hbm, negf_hbm, inemb_hbm, outemb_hbm,
             out_hbm, ci, pi, ni, cih, pih, nih, rc0, rc1, rp0, rp1,
             rn0, rn1, accv, sem0, sem1):
    rc = (rc0, rc1)
    rp = (rp0, rp1)
    rn = (rn0, rn1)
    wid = lax.axis_index("s") * NC + lax.axis_index("c")
    base = wid * BPW

    pltpu.sync_copy(center_hbm.at[pl.ds(base, BPW)], ci)
    pltpu.sync_copy(pos_hbm.at[pl.ds(base, BPW)], pi)
    pltpu.sync_copy(negf_hbm.at[pl.ds(base * K, BPW * K)], ni)

    def _halve(src, dst, n):
        def hbody(i, _):
            v = src[pl.ds(i * L, L)]
            dst[pl.ds(i * L, L)] = (
                lax.shift_left(lax.shift_right_logical(v, 13), 12)
                + (v & 4095))
            return 0
        lax.fori_loop(0, n // L, hbody, 0)

    _halve(ci, cih, BPW)
    _halve(pi, pih, BPW)
    _halve(ni, nih, BPW * K)

    sems = (sem0, sem1)

    def issue(c):
        s = c % 2
        hs = [
            pltpu.async_copy(inemb_hbm.at[cih.at[pl.ds(c * CH, CH)]],
                             rc[s], sems[s]),
            pltpu.async_copy(outemb_hbm.at[pih.at[pl.ds(c * CH, CH)]],
                             rp[s], sems[s]),
        ]
        for j in range(K):
            hs.append(pltpu.async_copy(
                outemb_hbm.at[nih.at[pl.ds(c * CH * K + j * CH, CH)]],
                rn[s].at[pl.ds(j * CH, CH)], sems[s]))
        return hs

    handles = {0: issue(0)}
    acc = jnp.zeros((L,), jnp.float32)

    for c in range(NCHUNK):
        if c + 1 < NCHUNK:
            handles[c + 1] = issue(c + 1)
        for h in handles.pop(c):
            h.wait()
        s = c % 2
        rc_s, rp_s, rn_s = rc[s], rp[s], rn[s]

        def gbody(g, acc):
            rows = g * L + lax.iota(jnp.int32, L)
            rows5 = rows * K
            abs_rows = c * CH + rows
            def _half(v):
                return lax.shift_left(
                    lax.shift_right_logical(v, 12) & 1, 6)

            parc = _half(plsc.load_gather(ci, [abs_rows]))
            parp = _half(plsc.load_gather(pi, [abs_rows]))
            pars = []
            for k in range(K):
                pars.append(_half(
                    plsc.load_gather(ni, [c * CH * K + rows5 + k])))

            def dbody(d, carry):
                pos, n0, n1, n2, n3, n4 = carry
                dv = jnp.full((L,), d, jnp.int32)
                cd = plsc.load_gather(rc_s, [rows, parc + dv])
                pd = plsc.load_gather(rp_s, [rows, parp + dv])
                pos = pos + cd * pd
                ns = []
                for k, nk in enumerate((n0, n1, n2, n3, n4)):
                    nd = plsc.load_gather(rn_s, [rows5 + k, pars[k] + dv])
                    ns.append(nk + cd * nd)
                return (pos, ns[0], ns[1], ns[2], ns[3], ns[4])

            z = jnp.zeros((L,), jnp.float32)
            pos, n0, n1, n2, n3, n4 = lax.fori_loop(
                0, D, dbody, (z, z, z, z, z, z))
            tot = _logsig(pos)
            for nk in (n0, n1, n2, n3, n4):
                tot = tot + _logsig(-nk)
            return acc - tot

        acc = lax.fori_loop(0, GRP, gbody, acc)

    accv[...] = acc
    pltpu.sync_copy(accv, out_hbm.at[wid // 8, pl.ds((wid % 8) * L, L)])


TRBI = 8192
TRO = TRBI // 2
NBLK = (VOCAB + TRBI - 1) // TRBI
PROWS = NBLK * TRO


def _transpose_body(a_ref, b_ref, oa_ref, ob_ref):
    oa_ref[:, 0:D] = a_ref[:, 0:TRO].T
    oa_ref[:, D:2 * D] = a_ref[:, TRO:TRBI].T
    ob_ref[:, 0:D] = b_ref[:, 0:TRO].T
    ob_ref[:, D:2 * D] = b_ref[:, TRO:TRBI].T


_transpose = pl.pallas_call(
    _transpose_body,
    grid=(NBLK,),
    in_specs=[
        pl.BlockSpec((D, TRBI), lambda i: (0, i)),
        pl.BlockSpec((D, TRBI), lambda i: (0, i)),
    ],
    out_specs=[
        pl.BlockSpec((TRO, 2 * D), lambda i: (i, 0)),
        pl.BlockSpec((TRO, 2 * D), lambda i: (i, 0)),
    ],
    out_shape=[
        jax.ShapeDtypeStruct((PROWS, 2 * D), jnp.float32),
        jax.ShapeDtypeStruct((PROWS, 2 * D), jnp.float32),
    ],
)


def _sum_body(x_ref, o_ref):
    o_ref[...] = jnp.full((1, 1), jnp.sum(x_ref[...]) * (1.0 / B),
                          jnp.float32)


_sum = pl.pallas_call(
    _sum_body,
    out_shape=jax.ShapeDtypeStruct((1, 1), jnp.float32),
)


def kernel(center, pos_context, neg_context, in_embed, out_embed):
    center = center.astype(jnp.int32)
    pos_context = pos_context.astype(jnp.int32)
    neg_flat = neg_context.astype(jnp.int32).reshape(-1)
    r_in, r_out = _transpose(in_embed.T, out_embed.T)
    partials = _sc_loss(center, pos_context, neg_flat, r_in, r_out)
    return _sum(partials)[0, 0]


# speedup vs baseline: 3.3748x; 1.2355x over previous
import functools

import jax
import jax.numpy as jnp
from jax import lax
from jax.experimental import pallas as pl
from jax.experimental.pallas import tpu as pltpu
from jax.experimental.pallas import tpu_sc as plsc

NC, NS, L = 2, 16, 16
NW = NC * NS
B = 16384
D = 64
VOCAB = 1000000
K = 5
BPW = B // NW
CH = 64
NCHUNK = BPW // CH
GRP = CH // L
DU = 16


def _logsig(x):
    e = jnp.exp(-jnp.abs(x))
    t = e / (2.0 + e)
    t2 = t * t
    p = 1.0 / 9.0
    for c in (1.0 / 7.0, 1.0 / 5.0, 1.0 / 3.0, 1.0):
        p = p * t2 + c
    return jnp.minimum(x, 0.0) - 2.0 * t * p


_mesh = plsc.VectorSubcoreMesh(core_axis_name="c", subcore_axis_name="s")


@functools.partial(
    pl.kernel,
    mesh=_mesh,
    compiler_params=pltpu.CompilerParams(
        needs_layout_passes=False, use_tc_tiling_on_sc=False),
    out_type=jax.ShapeDtypeStruct((NW // 8, 8 * L), jnp.float32),
    scratch_types=[
        pltpu.VMEM((BPW,), jnp.int32),
        pltpu.VMEM((BPW,), jnp.int32),
        pltpu.VMEM((BPW * K,), jnp.int32),
        pltpu.VMEM((BPW,), jnp.int32),
        pltpu.VMEM((BPW,), jnp.int32),
        pltpu.VMEM((BPW * K,), jnp.int32),
        pltpu.VMEM((CH, 2 * D), jnp.float32),
        pltpu.VMEM((CH, 2 * D), jnp.float32),
        pltpu.VMEM((CH, 2 * D), jnp.float32),
        pltpu.VMEM((CH, 2 * D), jnp.float32),
        pltpu.VMEM((CH * K, 2 * D), jnp.float32),
        pltpu.VMEM((CH * K, 2 * D), jnp.float32),
        pltpu.VMEM((L,), jnp.float32),
        pltpu.SemaphoreType.DMA,
        pltpu.SemaphoreType.DMA,
    ],
)
def _sc_loss(center_hbm, pos_hbm, negf_hbm, inemb_hbm, outemb_hbm,
             out_hbm, ci, pi, ni, cih, pih, nih, rc0, rc1, rp0, rp1,
             rn0, rn1, accv, sem0, sem1):
    rc = (rc0, rc1)
    rp = (rp0, rp1)
    rn = (rn0, rn1)
    wid = lax.axis_index("s") * NC + lax.axis_index("c")
    base = wid * BPW

    pltpu.sync_copy(center_hbm.at[pl.ds(base, BPW)], ci)
    pltpu.sync_copy(pos_hbm.at[pl.ds(base, BPW)], pi)
    pltpu.sync_copy(negf_hbm.at[pl.ds(base * K, BPW * K)], ni)

    def _halve(src, dst, n):
        def hbody(i, _):
            v = src[pl.ds(i * L, L)]
            dst[pl.ds(i * L, L)] = (
                lax.shift_left(lax.shift_right_logical(v, SH_I), SH_O)
                + (v & MSK))
            return 0
        lax.fori_loop(0, n // L, hbody, 0)

    _halve(ci, cih, BPW)
    _halve(pi, pih, BPW)
    _halve(ni, nih, BPW * K)

    sems = (sem0, sem1)

    def issue(c):
        s = c % 2
        hs = [
            pltpu.async_copy(inemb_hbm.at[cih.at[pl.ds(c * CH, CH)]],
                             rc[s], sems[s]),
            pltpu.async_copy(outemb_hbm.at[pih.at[pl.ds(c * CH, CH)]],
                             rp[s], sems[s]),
        ]
        for j in range(K):
            hs.append(pltpu.async_copy(
                outemb_hbm.at[nih.at[pl.ds(c * CH * K + j * CH, CH)]],
                rn[s].at[pl.ds(j * CH, CH)], sems[s]))
        return hs

    handles = {0: issue(0)}
    acc = jnp.zeros((L,), jnp.float32)

    for c in range(NCHUNK):
        if c + 1 < NCHUNK:
            handles[c + 1] = issue(c + 1)
        for h in handles.pop(c):
            h.wait()
        s = c % 2
        rc_s, rp_s, rn_s = rc[s], rp[s], rn[s]

        def gbody(g, acc):
            rows = g * L + lax.iota(jnp.int32, L)
            rows5 = rows * K
            abs_rows = c * CH + rows
            def _half(v):
                return lax.shift_left(
                    lax.shift_right_logical(v, SH_O) & 1, 6)

            parc = _half(plsc.load_gather(ci, [abs_rows]))
            parp = _half(plsc.load_gather(pi, [abs_rows]))
            pars = []
            for k in range(K):
                pars.append(_half(
                    plsc.load_gather(ni, [c * CH * K + rows5 + k])))

            rows5k = [rows5 + k for k in range(K)]

            def dbody(dbase, carry):
                pos, n0, n1, n2, n3, n4 = carry
                off = dbase * DU
                cb = parc + off
                pb = parp + off
                nb = [pars[k] + off for k in range(K)]
                ns = [n0, n1, n2, n3, n4]
                for dd in range(DU):
                    cd = plsc.load_gather(rc_s, [rows, cb + dd])
                    pd = plsc.load_gather(rp_s, [rows, pb + dd])
                    pos = pos + cd * pd
                    for k in range(K):
                        nd = plsc.load_gather(rn_s, [rows5k[k], nb[k] + dd])
                        ns[k] = ns[k] + cd * nd
                return (pos, ns[0], ns[1], ns[2], ns[3], ns[4])

            z = jnp.zeros((L,), jnp.float32)
            pos, n0, n1, n2, n3, n4 = lax.fori_loop(
                0, D // DU, dbody, (z, z, z, z, z, z))
            tot = _logsig(pos)
            for nk in (n0, n1, n2, n3, n4):
                tot = tot + _logsig(-nk)
            return acc - tot

        acc = lax.fori_loop(0, GRP, gbody, acc)

    accv[...] = acc
    pltpu.sync_copy(accv, out_hbm.at[wid // 8, pl.ds((wid % 8) * L, L)])


TRBI = 16384
TRO = TRBI // 2
NBLK = (VOCAB + TRBI - 1) // TRBI
PROWS = NBLK * TRO
SH_I = TRBI.bit_length() - 1
SH_O = TRO.bit_length() - 1
MSK = TRO - 1


def _transpose_body(a_ref, b_ref, oa_ref, ob_ref):
    oa_ref[...] = jnp.concatenate(
        [a_ref[:, 0:TRO], a_ref[:, TRO:TRBI]], axis=0).T
    ob_ref[...] = jnp.concatenate(
        [b_ref[:, 0:TRO], b_ref[:, TRO:TRBI]], axis=0).T


_transpose = pl.pallas_call(
    _transpose_body,
    grid=(NBLK,),
    in_specs=[
        pl.BlockSpec((D, TRBI), lambda i: (0, i)),
        pl.BlockSpec((D, TRBI), lambda i: (0, i)),
    ],
    out_specs=[
        pl.BlockSpec((TRO, 2 * D), lambda i: (i, 0)),
        pl.BlockSpec((TRO, 2 * D), lambda i: (i, 0)),
    ],
    out_shape=[
        jax.ShapeDtypeStruct((PROWS, 2 * D), jnp.float32),
        jax.ShapeDtypeStruct((PROWS, 2 * D), jnp.float32),
    ],
)


def _sum_body(x_ref, o_ref):
    o_ref[...] = jnp.full((1, 1), jnp.sum(x_ref[...]) * (1.0 / B),
                          jnp.float32)


_sum = pl.pallas_call(
    _sum_body,
    out_shape=jax.ShapeDtypeStruct((1, 1), jnp.float32),
)


def kernel(center, pos_context, neg_context, in_embed, out_embed):
    center = center.astype(jnp.int32)
    pos_context = pos_context.astype(jnp.int32)
    neg_flat = neg_context.astype(jnp.int32).reshape(-1)
    r_in, r_out = _transpose(in_embed.T, out_embed.T)
    partials = _sc_loss(center, pos_context, neg_flat, r_in, r_out)
    return _sum(partials)[0, 0]


# speedup vs baseline: 3.4205x; 1.0135x over previous
import functools

import jax
import jax.numpy as jnp
from jax import lax
from jax.experimental import pallas as pl
from jax.experimental.pallas import tpu as pltpu
from jax.experimental.pallas import tpu_sc as plsc

NC, NS, L = 2, 16, 16
NW = NC * NS
B = 16384
D = 64
VOCAB = 1000000
K = 5
BPW = B // NW
CH = 128
NCHUNK = BPW // CH
GRP = CH // L
DU = 16


def _logsig(x):
    e = jnp.exp(-jnp.abs(x))
    t = e / (2.0 + e)
    t2 = t * t
    p = 1.0 / 9.0
    for c in (1.0 / 7.0, 1.0 / 5.0, 1.0 / 3.0, 1.0):
        p = p * t2 + c
    return jnp.minimum(x, 0.0) - 2.0 * t * p


_mesh = plsc.VectorSubcoreMesh(core_axis_name="c", subcore_axis_name="s")


@functools.partial(
    pl.kernel,
    mesh=_mesh,
    compiler_params=pltpu.CompilerParams(
        needs_layout_passes=False, use_tc_tiling_on_sc=False),
    out_type=jax.ShapeDtypeStruct((NW // 8, 8 * L), jnp.float32),
    scratch_types=[
        pltpu.VMEM((BPW,), jnp.int32),
        pltpu.VMEM((BPW,), jnp.int32),
        pltpu.VMEM((BPW * K,), jnp.int32),
        pltpu.VMEM((BPW,), jnp.int32),
        pltpu.VMEM((BPW,), jnp.int32),
        pltpu.VMEM((BPW * K,), jnp.int32),
        pltpu.VMEM((CH, D), jnp.float32),
        pltpu.VMEM((CH, D), jnp.float32),
        pltpu.VMEM((CH, D), jnp.float32),
        pltpu.VMEM((CH, D), jnp.float32),
        pltpu.VMEM((CH * K, D), jnp.float32),
        pltpu.VMEM((CH * K, D), jnp.float32),
        pltpu.VMEM((L,), jnp.float32),
        pltpu.SemaphoreType.DMA,
        pltpu.SemaphoreType.DMA,
    ],
)
def _sc_loss(center_hbm, pos_hbm, negf_hbm, inemb_hbm, outemb_hbm,
             out_hbm, ci, pi, ni, cih, pih, nih, rc0, rc1, rp0, rp1,
             rn0, rn1, accv, sem0, sem1):
    rc = (rc0, rc1)
    rp = (rp0, rp1)
    rn = (rn0, rn1)
    wid = lax.axis_index("s") * NC + lax.axis_index("c")
    base = wid * BPW

    pltpu.sync_copy(center_hbm.at[pl.ds(base, BPW)], ci)
    pltpu.sync_copy(pos_hbm.at[pl.ds(base, BPW)], pi)
    pltpu.sync_copy(negf_hbm.at[pl.ds(base * K, BPW * K)], ni)

    def _remap(src, dst, n):
        def hbody(i, _):
            v = src[pl.ds(i * L, L)]
            pair = (lax.shift_left(lax.shift_right_logical(v, SH_I), SH_O)
                    + (v & MSK))
            half = lax.shift_right_logical(v, SH_O) & 1
            dst[pl.ds(i * L, L)] = lax.shift_left(pair, 1) + half
            return 0
        lax.fori_loop(0, n // L, hbody, 0)

    _remap(ci, cih, BPW)
    _remap(pi, pih, BPW)
    _remap(ni, nih, BPW * K)

    sems = (sem0, sem1)

    def issue(c):
        s = c % 2
        hs = [
            pltpu.async_copy(inemb_hbm.at[cih.at[pl.ds(c * CH, CH)]],
                             rc[s], sems[s]),
            pltpu.async_copy(outemb_hbm.at[pih.at[pl.ds(c * CH, CH)]],
                             rp[s], sems[s]),
        ]
        for j in range(K):
            hs.append(pltpu.async_copy(
                outemb_hbm.at[nih.at[pl.ds(c * CH * K + j * CH, CH)]],
                rn[s].at[pl.ds(j * CH, CH)], sems[s]))
        return hs

    handles = {0: issue(0)}
    acc = jnp.zeros((L,), jnp.float32)

    for c in range(NCHUNK):
        if c + 1 < NCHUNK:
            handles[c + 1] = issue(c + 1)
        for h in handles.pop(c):
            h.wait()
        s = c % 2
        rc_s, rp_s, rn_s = rc[s], rp[s], rn[s]

        def gbody(g, acc):
            rows = g * L + lax.iota(jnp.int32, L)
            rows5 = rows * K
            rows5k = [rows5 + k for k in range(K)]

            def dbody(dbase, carry):
                pos, n0, n1, n2, n3, n4 = carry
                off = jnp.full((L,), dbase * DU, jnp.int32)
                ns = [n0, n1, n2, n3, n4]
                for dd in range(DU):
                    col = off + dd
                    cd = plsc.load_gather(rc_s, [rows, col])
                    pd = plsc.load_gather(rp_s, [rows, col])
                    pos = pos + cd * pd
                    for k in range(K):
                        nd = plsc.load_gather(rn_s, [rows5k[k], col])
                        ns[k] = ns[k] + cd * nd
                return (pos, ns[0], ns[1], ns[2], ns[3], ns[4])

            z = jnp.zeros((L,), jnp.float32)
            pos, n0, n1, n2, n3, n4 = lax.fori_loop(
                0, D // DU, dbody, (z, z, z, z, z, z))
            tot = _logsig(pos)
            for nk in (n0, n1, n2, n3, n4):
                tot = tot + _logsig(-nk)
            return acc - tot

        acc = lax.fori_loop(0, GRP, gbody, acc)

    accv[...] = acc
    pltpu.sync_copy(accv, out_hbm.at[wid // 8, pl.ds((wid % 8) * L, L)])


TRBI = 16384
TRO = TRBI // 2
NBLK = (VOCAB + TRBI - 1) // TRBI
PROWS = NBLK * TRO
SH_I = TRBI.bit_length() - 1
SH_O = TRO.bit_length() - 1
MSK = TRO - 1


def _transpose_body(a_ref, b_ref, oa_ref, ob_ref):
    oa_ref[...] = jnp.concatenate(
        [a_ref[:, 0:TRO], a_ref[:, TRO:TRBI]], axis=0).T
    ob_ref[...] = jnp.concatenate(
        [b_ref[:, 0:TRO], b_ref[:, TRO:TRBI]], axis=0).T


_transpose = pl.pallas_call(
    _transpose_body,
    grid=(NBLK,),
    in_specs=[
        pl.BlockSpec((D, TRBI), lambda i: (0, i)),
        pl.BlockSpec((D, TRBI), lambda i: (0, i)),
    ],
    out_specs=[
        pl.BlockSpec((TRO, 2 * D), lambda i: (i, 0)),
        pl.BlockSpec((TRO, 2 * D), lambda i: (i, 0)),
    ],
    out_shape=[
        jax.ShapeDtypeStruct((PROWS, 2 * D), jnp.float32),
        jax.ShapeDtypeStruct((PROWS, 2 * D), jnp.float32),
    ],
)


def _sum_body(x_ref, o_ref):
    o_ref[...] = jnp.full((1, 1), jnp.sum(x_ref[...]) * (1.0 / B),
                          jnp.float32)


_sum = pl.pallas_call(
    _sum_body,
    out_shape=jax.ShapeDtypeStruct((1, 1), jnp.float32),
)


def kernel(center, pos_context, neg_context, in_embed, out_embed):
    center = center.astype(jnp.int32)
    pos_context = pos_context.astype(jnp.int32)
    neg_flat = neg_context.astype(jnp.int32).reshape(-1)
    r_in, r_out = _transpose(in_embed.T, out_embed.T)
    r_in = r_in.reshape(2 * PROWS, D)
    r_out = r_out.reshape(2 * PROWS, D)
    partials = _sc_loss(center, pos_context, neg_flat, r_in, r_out)
    return _sum(partials)[0, 0]
